# baseline (speedup 1.0000x reference)
import jax
import jax.numpy as jnp
from jax import lax
from jax.experimental import pallas as pl
from jax.experimental.pallas import tpu as pltpu
from jax.experimental.pallas import tpu_sc as plsc

B, S = 4, 4096
D_MODEL = 1024
N_TOK = B * S

_info = plsc.get_sparse_core_info()
NC, NS = _info.num_cores, _info.num_subcores
NW = NC * NS
ROWS_PER_W = N_TOK // NW
W_PER_ROW = S // ROWS_PER_W
GC = 16
K = 2
NBUF = 6
NG = ROWS_PER_W // GC
NO = NG // K


def _emb_kernel(table_hbm, idx_hbm, out_hbm, idx_v, ring, *sems):
    gsems = sems[:NBUF]
    osems = sems[NBUF:]
    wid = lax.axis_index("s") * NC + lax.axis_index("c")
    pltpu.sync_copy(
        idx_hbm.at[wid // W_PER_ROW,
                   pl.ds((wid % W_PER_ROW) * ROWS_PER_W, ROWS_PER_W)],
        idx_v)
    base = wid * ROWS_PER_W // GC

    def gather(g):
        b = g % NBUF
        return pltpu.async_copy(
            table_hbm.at[idx_v.at[pl.ds(g * GC, GC)]], ring.at[b], gsems[b])

    gathers = [None] * NG
    outs = [None] * NO
    for h in range(K * (NBUF // K - 1)):
        gathers[h] = gather(h)
    for o in range(NO):
        for j in range(K):
            gathers[o * K + j].wait()
        outs[o] = pltpu.async_copy(
            ring.at[pl.ds((o * K) % NBUF, K)],
            out_hbm.at[pl.ds(base + o * K, K)], osems[o % (NBUF // K)])
        ho = o + NBUF // K - 1
        if ho < NO:
            if o >= 1:
                outs[o - 1].wait()
            for j in range(K):
                gathers[ho * K + j] = gather(ho * K + j)
    for o in range(max(0, NO - NBUF // K), NO):
        if outs[o] is not None:
            outs[o].wait()


def kernel(tokens, W_E):
    mesh = plsc.VectorSubcoreMesh(core_axis_name="c", subcore_axis_name="s")
    scratch = (
        [pltpu.VMEM((ROWS_PER_W,), jnp.int32),
         pltpu.VMEM((NBUF, GC, D_MODEL), jnp.float32)]
        + [pltpu.SemaphoreType.DMA for _ in range(NBUF + NBUF // K)]
    )
    out = pl.kernel(
        _emb_kernel,
        mesh=mesh,
        out_type=jax.ShapeDtypeStruct((N_TOK // GC, GC, D_MODEL), jnp.float32),
        scratch_types=scratch,
    )(W_E, tokens)
    return out.reshape(B, S, D_MODEL)
